# baseline (speedup 1.0000x reference)
import functools

import jax
import jax.numpy as jnp
from jax import lax
from jax.experimental import pallas as pl
from jax.experimental.pallas import tpu as pltpu
from jax.experimental.pallas import tpu_sc as plsc

N_EXPERTS = 64
TOPK = 8
BLOCK_COLS = 1024
NC = 2
NS = 16
NW = NC * NS


def _score_kernel(x_ref, wt_ref, b_ref, key_ref):
    s = jnp.dot(x_ref[...], wt_ref[...], preferred_element_type=jnp.float32)
    s = s + b_ref[...]
    iota = lax.broadcasted_iota(jnp.int32, s.shape, 1)
    b = lax.bitcast_convert_type(s, jnp.int32)
    low = jnp.where(b >= 0, (N_EXPERTS - 1) - iota, iota)
    key = lax.bitcast_convert_type((b & ~(N_EXPERTS - 1)) | low, jnp.float32)
    key_ref[...] = key.T


def _unpack(key):
    b = lax.bitcast_convert_type(key, jnp.int32)
    low = b & (N_EXPERTS - 1)
    idx = jnp.where(b >= 0, (N_EXPERTS - 1) - low, low)
    val = lax.bitcast_convert_type(b & ~(N_EXPERTS - 1), jnp.float32)
    return val, idx


def _bitonic_merge(vals, descending):
    n = len(vals)
    if n == 1:
        return vals
    half = n // 2
    lo, hi = [], []
    for i in range(half):
        a, b = vals[i], vals[i + half]
        if descending:
            lo.append(jnp.maximum(a, b))
            hi.append(jnp.minimum(a, b))
        else:
            lo.append(jnp.minimum(a, b))
            hi.append(jnp.maximum(a, b))
    return (_bitonic_merge(lo, descending) + _bitonic_merge(hi, descending))


def _bitonic_sort(vals, descending):
    n = len(vals)
    if n == 1:
        return vals
    half = n // 2
    first = _bitonic_sort(vals[:half], True)
    second = _bitonic_sort(vals[half:], False)
    return _bitonic_merge(first + second, descending)


def _bitonic_sort8(vals):
    return _bitonic_sort(vals, True)


def _merge_top8(a, b):
    c = [jnp.maximum(a[i], b[TOPK - 1 - i]) for i in range(TOPK)]
    return _bitonic_merge(c, True)


def _topk_sc_kernel(keys_hbm, w_hbm, i_hbm, keys_v, w_v, i_v, sem0, sem1):
    rows = keys_hbm.shape[1] // NW
    groups = rows // 16
    wid = lax.axis_index("s") * NC + lax.axis_index("c")
    base = wid * rows
    half = rows // 2
    cp0 = pltpu.async_copy(keys_hbm.at[:, pl.ds(base, half)],
                           keys_v.at[:, pl.ds(0, half)], sem0)
    cp1 = pltpu.async_copy(keys_hbm.at[:, pl.ds(base + half, half)],
                           keys_v.at[:, pl.ds(half, half)], sem1)

    def body(g, carry):
        sl = pl.ds(g * 16, 16)
        cur = [keys_v[e, sl] for e in range(N_EXPERTS)]
        sorted8 = [_bitonic_sort8(cur[b * 8:(b + 1) * 8]) for b in range(8)]
        while len(sorted8) > 1:
            sorted8 = [_merge_top8(sorted8[i], sorted8[i + 1])
                       for i in range(0, len(sorted8), 2)]
        tops = sorted8[0]
        vals, idxs = zip(*(_unpack(t) for t in tops))
        es = [jnp.exp(v - vals[0]) for v in vals]
        tot = es[0]
        for k in range(1, TOPK):
            tot = tot + es[k]
        for k in range(TOPK):
            w_v[k, sl] = es[k] / tot
            i_v[k, sl] = idxs[k]
        return carry

    cp0.wait()
    lax.fori_loop(0, groups // 2, body, 0)
    cp1.wait()
    lax.fori_loop(groups // 2, groups, body, 0)
    pltpu.sync_copy(w_v, w_hbm.at[:, pl.ds(base, rows)])
    pltpu.sync_copy(i_v, i_hbm.at[:, pl.ds(base, rows)])


N_CHUNKS = 1


def kernel(x, weight, bias, target_dist):
    del target_dist
    n_tokens, dim = x.shape
    wt = weight.T
    b2 = bias.reshape(1, N_EXPERTS)
    chunk = n_tokens // N_CHUNKS
    grid = (chunk // BLOCK_COLS,)
    score_call = pl.pallas_call(
        _score_kernel,
        grid=grid,
        in_specs=[
            pl.BlockSpec((BLOCK_COLS, dim), lambda i: (i, 0)),
            pl.BlockSpec((dim, N_EXPERTS), lambda i: (0, 0)),
            pl.BlockSpec((1, N_EXPERTS), lambda i: (0, 0)),
        ],
        out_specs=pl.BlockSpec((N_EXPERTS, BLOCK_COLS), lambda i: (0, i)),
        out_shape=jax.ShapeDtypeStruct((N_EXPERTS, chunk), jnp.float32),
    )

    rows = chunk // NW
    sc_topk = functools.partial(
        pl.kernel,
        mesh=plsc.VectorSubcoreMesh(core_axis_name="c", subcore_axis_name="s"),
        out_type=[
            jax.ShapeDtypeStruct((TOPK, chunk), jnp.float32),
            jax.ShapeDtypeStruct((TOPK, chunk), jnp.int32),
        ],
        scratch_types=[
            pltpu.VMEM((N_EXPERTS, rows), jnp.float32),
            pltpu.VMEM((TOPK, rows), jnp.float32),
            pltpu.VMEM((TOPK, rows), jnp.int32),
            pltpu.SemaphoreType.DMA,
            pltpu.SemaphoreType.DMA,
        ],
    )(_topk_sc_kernel)

    parts = []
    for c in range(N_CHUNKS):
        keys_t = score_call(
            jax.lax.slice_in_dim(x, c * chunk, (c + 1) * chunk), wt, b2)
        parts.append(sc_topk(keys_t))
    w_t = jnp.concatenate([p[0] for p in parts], axis=1)
    i_t = jnp.concatenate([p[1] for p in parts], axis=1)
    return (w_t.T, i_t.T)


# speedup vs baseline: 1.0118x; 1.0118x over previous
import functools

import jax
import jax.numpy as jnp
from jax import lax
from jax.experimental import pallas as pl
from jax.experimental.pallas import tpu as pltpu
from jax.experimental.pallas import tpu_sc as plsc

N_EXPERTS = 64
TOPK = 8
BLOCK_COLS = 1024
NC = 2
NS = 16
NW = NC * NS


def _score_kernel(x_ref, wt_ref, b_ref, key_ref):
    s = jnp.dot(x_ref[...], wt_ref[...], preferred_element_type=jnp.float32)
    s = s + b_ref[...]
    iota = lax.broadcasted_iota(jnp.int32, s.shape, 1)
    b = lax.bitcast_convert_type(s, jnp.int32)
    low = jnp.where(b >= 0, (N_EXPERTS - 1) - iota, iota)
    key = lax.bitcast_convert_type((b & ~(N_EXPERTS - 1)) | low, jnp.float32)
    key_ref[...] = key.T


def _unpack(key):
    b = lax.bitcast_convert_type(key, jnp.int32)
    low = b & (N_EXPERTS - 1)
    idx = jnp.where(b >= 0, (N_EXPERTS - 1) - low, low)
    val = lax.bitcast_convert_type(b & ~(N_EXPERTS - 1), jnp.float32)
    return val, idx


def _bitonic_merge(vals, descending):
    n = len(vals)
    if n == 1:
        return vals
    half = n // 2
    lo, hi = [], []
    for i in range(half):
        a, b = vals[i], vals[i + half]
        if descending:
            lo.append(jnp.maximum(a, b))
            hi.append(jnp.minimum(a, b))
        else:
            lo.append(jnp.minimum(a, b))
            hi.append(jnp.maximum(a, b))
    return (_bitonic_merge(lo, descending) + _bitonic_merge(hi, descending))


def _bitonic_sort(vals, descending):
    n = len(vals)
    if n == 1:
        return vals
    half = n // 2
    first = _bitonic_sort(vals[:half], True)
    second = _bitonic_sort(vals[half:], False)
    return _bitonic_merge(first + second, descending)


def _bitonic_sort8(vals):
    return _bitonic_sort(vals, True)


def _merge_top8(a, b):
    c = [jnp.maximum(a[i], b[TOPK - 1 - i]) for i in range(TOPK)]
    return _bitonic_merge(c, True)


def _topk_sc_kernel(keys_hbm, w_hbm, i_hbm, keys_v, w_v, i_v):
    rows = keys_hbm.shape[1] // NW
    groups = rows // 16
    wid = lax.axis_index("s") * NC + lax.axis_index("c")
    base = wid * rows
    pltpu.sync_copy(keys_hbm.at[:, pl.ds(base, rows)], keys_v)

    def body(g, carry):
        sl = pl.ds(g * 16, 16)
        cur = [keys_v[e, sl] for e in range(N_EXPERTS)]
        sorted8 = [_bitonic_sort8(cur[b * 8:(b + 1) * 8]) for b in range(8)]
        while len(sorted8) > 1:
            sorted8 = [_merge_top8(sorted8[i], sorted8[i + 1])
                       for i in range(0, len(sorted8), 2)]
        tops = sorted8[0]
        vals, idxs = zip(*(_unpack(t) for t in tops))
        es = [jnp.exp(v - vals[0]) for v in vals]
        tot = es[0]
        for k in range(1, TOPK):
            tot = tot + es[k]
        for k in range(TOPK):
            w_v[k, sl] = es[k] / tot
            i_v[k, sl] = idxs[k]
        return carry

    lax.fori_loop(0, groups, body, 0)
    pltpu.sync_copy(w_v, w_hbm.at[:, pl.ds(base, rows)])
    pltpu.sync_copy(i_v, i_hbm.at[:, pl.ds(base, rows)])


N_CHUNKS = 1


def kernel(x, weight, bias, target_dist):
    del target_dist
    n_tokens, dim = x.shape
    wt = weight.T
    b2 = bias.reshape(1, N_EXPERTS)
    chunk = n_tokens // N_CHUNKS
    grid = (chunk // BLOCK_COLS,)
    score_call = pl.pallas_call(
        _score_kernel,
        grid=grid,
        in_specs=[
            pl.BlockSpec((BLOCK_COLS, dim), lambda i: (i, 0)),
            pl.BlockSpec((dim, N_EXPERTS), lambda i: (0, 0)),
            pl.BlockSpec((1, N_EXPERTS), lambda i: (0, 0)),
        ],
        out_specs=pl.BlockSpec((N_EXPERTS, BLOCK_COLS), lambda i: (0, i)),
        out_shape=jax.ShapeDtypeStruct((N_EXPERTS, chunk), jnp.float32),
    )

    rows = chunk // NW
    sc_topk = functools.partial(
        pl.kernel,
        mesh=plsc.VectorSubcoreMesh(core_axis_name="c", subcore_axis_name="s"),
        out_type=[
            jax.ShapeDtypeStruct((TOPK, chunk), jnp.float32),
            jax.ShapeDtypeStruct((TOPK, chunk), jnp.int32),
        ],
        scratch_types=[
            pltpu.VMEM((N_EXPERTS, rows), jnp.float32),
            pltpu.VMEM((TOPK, rows), jnp.float32),
            pltpu.VMEM((TOPK, rows), jnp.int32),
        ],
    )(_topk_sc_kernel)

    parts = []
    for c in range(N_CHUNKS):
        keys_t = score_call(
            jax.lax.slice_in_dim(x, c * chunk, (c + 1) * chunk), wt, b2)
        parts.append(sc_topk(keys_t))
    w_t = jnp.concatenate([p[0] for p in parts], axis=1)
    i_t = jnp.concatenate([p[1] for p in parts], axis=1)
    return (w_t.T, i_t.T)
